# baseline (speedup 1.0000x reference)
import functools
import math

import jax
import jax.numpy as jnp
from jax import lax
from jax.experimental import pallas as pl
from jax.experimental.pallas import tpu as pltpu

KNN_K = 20
SIREN_OMEGA = 30.0
N_FOURIER = 10

USE_PALLAS_DIST = True
USE_AB_SPLIT = False


def _leaky(z):
    return jnp.where(z >= 0, z, 0.2 * z)


def _gn(x, gamma, beta):
    axes = tuple(range(1, x.ndim))
    mean = jnp.mean(x, axis=axes, keepdims=True)
    var = jnp.var(x, axis=axes, keepdims=True)
    xn = (x - mean) / jnp.sqrt(var + 1e-5)
    shape = (1, -1) + (1,) * (x.ndim - 2)
    return xn * gamma.reshape(shape) + beta.reshape(shape)


def _dist_body(x_row_ref, x_all_ref, out_ref):
    xr = x_row_ref[0]
    xa = x_all_ref[0]
    inner = jax.lax.dot_general(xr, xa, (((0,), (0,)), ((), ())),
                                preferred_element_type=jnp.float32)
    xx_r = jnp.sum(xr * xr, axis=0)
    xx_a = jnp.sum(xa * xa, axis=0)
    out_ref[0] = 2.0 * inner - xx_r[:, None] - xx_a[None, :]


def _neg_dist_pallas(x):
    B, C, N = x.shape
    RT = 512
    return pl.pallas_call(
        _dist_body,
        grid=(B, N // RT),
        in_specs=[
            pl.BlockSpec((1, C, RT), lambda b, r: (b, 0, r)),
            pl.BlockSpec((1, C, N), lambda b, r: (b, 0, 0)),
        ],
        out_specs=pl.BlockSpec((1, RT, N), lambda b, r: (b, r, 0)),
        out_shape=jax.ShapeDtypeStruct((B, N, N), jnp.float32),
    )(x, x)


def _neg_dist_jnp(x):
    inner = jnp.einsum('bcn,bcm->bnm', x, x)
    xx = jnp.sum(x * x, axis=1)
    return 2.0 * inner - xx[:, :, None] - xx[:, None, :]


def _edge_conv_ref(x, W, gamma, beta):
    B, C, N = x.shape
    scores = _neg_dist_pallas(x) if USE_PALLAS_DIST else _neg_dist_jnp(x)
    idx = lax.top_k(scores, KNN_K)[1]
    x_t = jnp.transpose(x, (0, 2, 1))
    nbr = jax.vmap(lambda xt, id_: xt[id_])(x_t, idx)
    centre = jnp.broadcast_to(x_t[:, :, None, :], (B, N, KNN_K, C))
    edge = jnp.concatenate([centre, nbr - centre], axis=-1)
    e = jnp.transpose(edge, (0, 3, 1, 2))
    h = jnp.einsum('oc,bcnk->bonk', W, e)
    h = _gn(h, gamma, beta)
    h = _leaky(h)
    return jnp.max(h, axis=-1)


def _edge_conv_ab(x, W, gamma, beta):
    B, C, N = x.shape
    O = W.shape[0]
    W1 = W[:, :C]
    W2 = W[:, C:]
    a = jnp.einsum('oc,bcn->bon', W1 - W2, x)
    b = jnp.einsum('oc,bcn->bon', W2, x)

    scores = _neg_dist_pallas(x) if USE_PALLAS_DIST else _neg_dist_jnp(x)
    idx = lax.top_k(scores, KNN_K)[1]

    b_t = jnp.transpose(b, (0, 2, 1))
    g = jax.vmap(lambda bt, id_: bt[id_])(b_t, idx)
    maxb = jnp.max(g, axis=2)
    Sb = jnp.sum(g, axis=2)
    Qb = jnp.sum(g * g, axis=2)

    cnt = O * N * KNN_K
    a_t = jnp.transpose(a, (0, 2, 1))
    s1 = KNN_K * jnp.sum(a_t, axis=(1, 2)) + jnp.sum(Sb, axis=(1, 2))
    s2 = (KNN_K * jnp.sum(a_t * a_t, axis=(1, 2))
          + 2.0 * jnp.sum(a_t * Sb, axis=(1, 2))
          + jnp.sum(Qb, axis=(1, 2)))
    mean = s1 / cnt
    var = s2 / cnt - mean * mean
    rstd = 1.0 / jnp.sqrt(var + 1e-5)

    hmax = a_t + maxb
    xn = (hmax - mean[:, None, None]) * rstd[:, None, None]
    out = _leaky(xn * gamma[None, None, :] + beta[None, None, :])
    return jnp.transpose(out, (0, 2, 1))


def kernel(x, cond, qxyz, params):
    p = params
    ec = _edge_conv_ab if USE_AB_SPLIT else _edge_conv_ref
    x1 = ec(x, p['ec1_W'], p['ec1_g'], p['ec1_b'])
    x2 = ec(x1, p['ec2_W'], p['ec2_g'], p['ec2_b'])
    x3 = ec(x2, p['ec3_W'], p['ec3_g'], p['ec3_b'])
    cat = jnp.concatenate([x1, x2, x3], axis=1)
    h = jnp.einsum('oc,bcn->bon', p['fuse_W'], cat)
    h = _gn(h, p['fuse_g'], p['fuse_b'])
    h = _leaky(h)
    g_geo = jnp.max(h, axis=-1)
    c = jax.nn.relu(cond @ p['c1_W'].T + p['c1_b'])
    c = c @ p['c2_W'].T + p['c2_b']
    g = jnp.concatenate([g_geo, c], axis=1)
    s = _leaky(g @ p['s1_W'].T + p['s1_b'])
    scalars = s @ p['s2_W'].T + p['s2_b']

    Bq = qxyz.shape[0]
    freqs = jnp.pi * (2.0 ** jnp.arange(N_FOURIER, dtype=jnp.float32))
    xs = qxyz[..., None] * freqs.reshape(1, 1, 1, -1)
    sins = jnp.sin(xs).reshape(Bq, 3 * N_FOURIER, -1)
    coss = jnp.cos(xs).reshape(Bq, 3 * N_FOURIER, -1)
    q = jnp.concatenate([qxyz, sins, coss], axis=1)

    def film(xq, W, b, fW, fb):
        hh = SIREN_OMEGA * (jnp.einsum('oc,bcn->bon', W, xq) + b[None, :, None])
        bias = (g @ fW.T + fb)[:, :, None]
        return jnp.sin(hh + bias)

    q = film(q, p['f1_W'], p['f1_b'], p['f1_fW'], p['f1_fb'])
    q = film(q, p['f2_W'], p['f2_b'], p['f2_fW'], p['f2_fb'])
    field = jnp.einsum('oc,bcn->bon', p['fo_W'], q) + p['fo_b'][None, :, None]
    return (scalars, field)


# speedup vs baseline: 5.3836x; 5.3836x over previous
import functools
import math

import jax
import jax.numpy as jnp
from jax import lax
from jax.experimental import pallas as pl
from jax.experimental.pallas import tpu as pltpu
from jax.experimental.pallas import tpu_sc as plsc

KNN_K = 20
SIREN_OMEGA = 30.0
N_FOURIER = 10
NEG = float('-inf')

try:
    _SC = plsc.get_sparse_core_info()
    _NC, _NS = _SC.num_cores, _SC.num_subcores
except Exception:
    _NC, _NS = 2, 16
_NW = _NC * _NS


def _leaky(z):
    return jnp.where(z >= 0, z, 0.2 * z)


def _dist_body(x_row_ref, x_all_ref, out_ref):
    xr = x_row_ref[0]
    xa = x_all_ref[0]
    inner = jax.lax.dot_general(xr, xa, (((0,), (0,)), ((), ())),
                                preferred_element_type=jnp.float32)
    xx_r = jnp.sum(xr * xr, axis=0)
    xx_a = jnp.sum(xa * xa, axis=0)
    out_ref[0] = 2.0 * inner - xx_r[:, None] - xx_a[None, :]


def _neg_dist(x):
    B, C, N = x.shape
    RT = 512
    return pl.pallas_call(
        _dist_body,
        grid=(B, N // RT),
        in_specs=[
            pl.BlockSpec((1, C, RT), lambda b, r: (b, 0, r)),
            pl.BlockSpec((1, C, N), lambda b, r: (b, 0, 0)),
        ],
        out_specs=pl.BlockSpec((1, RT, N), lambda b, r: (b, r, 0)),
        out_shape=jax.ShapeDtypeStruct((B, N, N), jnp.float32),
    )(x, x)


def _lane_splat_f32(vec, lane, iota):
    m = jnp.where(iota == lane, vec, jnp.full((16,), NEG, jnp.float32))
    return jnp.full((16,), jnp.max(m))


NBR = 24


@functools.partial(jax.jit, static_argnames=('n_pts', 'cp'))
def _sc_topk_gather(scores, xt, n_pts, cp):
    BN, N = scores.shape
    K = KNN_K
    rows_per_w = BN // _NW
    mesh = plsc.VectorSubcoreMesh(core_axis_name="c", subcore_axis_name="s")

    @functools.partial(
        pl.kernel,
        out_type=jax.ShapeDtypeStruct((BN, NBR, cp), jnp.float32),
        mesh=mesh,
        compiler_params=pltpu.CompilerParams(needs_layout_passes=False,
                                             use_tc_tiling_on_sc=False),
        scratch_types=[
            pltpu.VMEM((1, N), jnp.float32),
            pltpu.VMEM((N,), jnp.int32),
            pltpu.VMEM((NBR,), jnp.int32),
            pltpu.VMEM((NBR, cp), jnp.float32),
            pltpu.SemaphoreType.DMA,
        ],
    )
    def body(scores_hbm, xt_hbm, nbr_hbm, row_v, cand_v, sel_v, rows_v, sem):
        wid = lax.axis_index("s") * _NC + lax.axis_index("c")
        base = wid * rows_per_w
        bN = (base // n_pts) * n_pts
        iota = lax.iota(jnp.int32, 16)
        nvecs = N // 16
        neg_v = jnp.full((16,), NEG, jnp.float32)

        def do_row(r, _carry):
            i = base + r
            pltpu.sync_copy(scores_hbm.at[pl.ds(i, 1)], row_v)

            def amax(h):
                def st(j, acc):
                    return jnp.maximum(
                        acc, row_v[0, pl.ds(h * (N // 2) + j * 16, 16)])
                return lax.fori_loop(0, nvecs // 2, st, neg_v)
            acc0 = amax(0)
            acc1 = amax(1)
            s0, _ = plsc.sort_key_val(acc0, iota, descending=True)
            s1, _ = plsc.sort_key_val(acc1, iota, descending=True)
            hmin = jnp.minimum(s0, jnp.flip(s1, 0))
            sh, _ = plsc.sort_key_val(hmin, iota, descending=True)
            t_lb = _lane_splat_f32(sh, 3, iota)

            def bstep(j, off):
                v = row_v[0, pl.ds(j * 16, 16)]
                m = v >= t_lb
                csum = plsc.cumsum(m.astype(jnp.int32))
                pos = off + csum - 1
                plsc.store_scatter(cand_v, [pos], j * 16 + iota, mask=m)
                return off + plsc.all_reduce_population_count(m)
            offv = lax.fori_loop(0, nvecs, bstep,
                                 jnp.zeros((16,), jnp.int32))
            ncand = jnp.max(offv)
            nvec_c = (ncand + 15) // 16
            ncand_v = jnp.full((16,), ncand)
            zero16 = jnp.zeros((16,), jnp.int32)

            def load_cand(v):
                idxs = cand_v[pl.ds(v * 16, 16)]
                ok = (v * 16 + iota) < ncand_v
                idxs = jnp.where(ok, idxs, zero16)
                vals = plsc.load_gather(row_v, [zero16, idxs])
                return idxs, jnp.where(ok, vals, neg_v)

            def cstep(v, carry):
                T0, T1 = carry
                _, vals = load_cand(v)
                s, _ = plsc.sort_key_val(vals, iota, descending=True)
                u0 = jnp.maximum(T1, jnp.flip(s, 0))
                su, _ = plsc.sort_key_val(u0, iota, descending=True)
                ru = jnp.flip(su, 0)
                n0, _ = plsc.sort_key_val(jnp.maximum(T0, ru), iota,
                                          descending=True)
                n1, _ = plsc.sort_key_val(jnp.minimum(T0, ru), iota,
                                          descending=True)
                return n0, n1
            _, T1 = lax.fori_loop(0, nvec_c, cstep, (neg_v, neg_v))
            t20 = _lane_splat_f32(T1, 3, iota)

            self_v = jnp.full((16,), i)
            sel_v[pl.ds(0, 16)] = self_v
            sel_v[pl.ds(NBR - 16, 16)] = self_v
            cap = jnp.full((16,), NBR)

            def d1(v, cnt):
                idxs, vals = load_cand(v)
                m = vals > t20
                pos = cnt + plsc.cumsum(m.astype(jnp.int32)) - 1
                keep = m & (pos < cap)
                plsc.store_scatter(sel_v, [pos], bN + idxs, mask=keep)
                return cnt + plsc.all_reduce_population_count(keep)
            ngt = lax.fori_loop(0, nvec_c, d1, jnp.zeros((16,), jnp.int32))

            def d2(v, cnt):
                idxs, vals = load_cand(v)
                m = vals == t20
                c = cnt + plsc.cumsum(m.astype(jnp.int32))
                keep = m & (c <= K)
                plsc.store_scatter(sel_v, [c - 1], bN + idxs, mask=keep)
                return cnt + plsc.all_reduce_population_count(keep)
            lax.fori_loop(0, nvec_c, d2, ngt)

            pltpu.async_copy(xt_hbm.at[sel_v], rows_v, sem).wait()
            pltpu.sync_copy(rows_v, nbr_hbm.at[i])
            return _carry

        lax.fori_loop(0, rows_per_w, do_row, jnp.int32(0))

    return body(scores, xt)


def _conv_body(nbr_ref, wa_ref, wb_ref, hmax_ref, s1_ref, s2_ref):
    xi = nbr_ref[:, KNN_K, :]
    a = jax.lax.dot_general(xi, wa_ref[...], (((1,), (0,)), ((), ())),
                            preferred_element_type=jnp.float32)
    wb = wb_ref[...]
    acc = None
    s1 = jnp.float32(0.0)
    s2 = jnp.float32(0.0)
    for kk in range(KNN_K):
        h = a + jax.lax.dot_general(
            nbr_ref[:, kk, :] - xi, wb, (((1,), (0,)), ((), ())),
            preferred_element_type=jnp.float32)
        s1 = s1 + jnp.sum(h)
        s2 = s2 + jnp.sum(h * h)
        acc = h if acc is None else jnp.maximum(acc, h)
    hmax_ref[...] = acc
    s1_ref[...] = s1.reshape(1, 1, 1)
    s2_ref[...] = s2.reshape(1, 1, 1)


def _conv(nbr, wa, wb):
    BN, NB, Cp = nbr.shape
    O = wa.shape[1]
    P = 512
    G = BN // P
    return pl.pallas_call(
        _conv_body,
        grid=(G,),
        in_specs=[
            pl.BlockSpec((P, NB, Cp), lambda g: (g, 0, 0)),
            pl.BlockSpec((Cp, O), lambda g: (0, 0)),
            pl.BlockSpec((Cp, O), lambda g: (0, 0)),
        ],
        out_specs=[
            pl.BlockSpec((P, O), lambda g: (g, 0)),
            pl.BlockSpec((1, 1, 1), lambda g: (g, 0, 0)),
            pl.BlockSpec((1, 1, 1), lambda g: (g, 0, 0)),
        ],
        out_shape=[
            jax.ShapeDtypeStruct((BN, O), jnp.float32),
            jax.ShapeDtypeStruct((G, 1, 1), jnp.float32),
            jax.ShapeDtypeStruct((G, 1, 1), jnp.float32),
        ],
    )(nbr, wa, wb)


def _norm_body(h_ref, mu_ref, den_ref, g_ref, b_ref, o_ref):
    mu = mu_ref[0]
    den = den_ref[0]
    xn = (h_ref[0] - mu) / den
    o_ref[0] = _leaky(xn * g_ref[...] + b_ref[...])


def _norm(h, mu, den, gamma, beta, B):
    BN, O = h.shape
    N = BN // B
    P = 1024
    h3 = h.reshape(B, N, O)
    return pl.pallas_call(
        _norm_body,
        grid=(B, N // P),
        in_specs=[
            pl.BlockSpec((1, P, O), lambda b, g: (b, g, 0)),
            pl.BlockSpec((1, 1, 1), lambda b, g: (b, 0, 0)),
            pl.BlockSpec((1, 1, 1), lambda b, g: (b, 0, 0)),
            pl.BlockSpec((1, O), lambda b, g: (0, 0)),
            pl.BlockSpec((1, O), lambda b, g: (0, 0)),
        ],
        out_specs=pl.BlockSpec((1, P, O), lambda b, g: (b, g, 0)),
        out_shape=jax.ShapeDtypeStruct((B, N, O), jnp.float32),
    )(h3, mu.reshape(B, 1, 1), den.reshape(B, 1, 1),
      gamma.reshape(1, O), beta.reshape(1, O))


def _fuse_body(c_ref, w_ref, m_ref, s1_ref, s2_ref):
    h = jax.lax.dot_general(c_ref[...], w_ref[...], (((1,), (0,)), ((), ())),
                            preferred_element_type=jnp.float32)
    m_ref[...] = jnp.max(h, axis=0).reshape(1, 1, -1)
    s1_ref[...] = jnp.sum(h).reshape(1, 1, 1)
    s2_ref[...] = jnp.sum(h * h).reshape(1, 1, 1)


def _fuse(cat, wt):
    BN, C = cat.shape
    O = wt.shape[1]
    P = 512
    G = BN // P
    return pl.pallas_call(
        _fuse_body,
        grid=(G,),
        in_specs=[
            pl.BlockSpec((P, C), lambda g: (g, 0)),
            pl.BlockSpec((C, O), lambda g: (0, 0)),
        ],
        out_specs=[
            pl.BlockSpec((1, 1, O), lambda g: (g, 0, 0)),
            pl.BlockSpec((1, 1, 1), lambda g: (g, 0, 0)),
            pl.BlockSpec((1, 1, 1), lambda g: (g, 0, 0)),
        ],
        out_shape=[
            jax.ShapeDtypeStruct((G, 1, O), jnp.float32),
            jax.ShapeDtypeStruct((G, 1, 1), jnp.float32),
            jax.ShapeDtypeStruct((G, 1, 1), jnp.float32),
        ],
    )(cat, wt)


def _heads_body(m_ref, muf_ref, denf_ref, fg_ref, fb_ref, cond_ref,
                c1t_ref, c1b_ref, c2t_ref, c2b_ref,
                s1t_ref, s1b_ref, s2t_ref, s2b_ref,
                q_ref, f1t_ref, f1b_ref, f1ft_ref, f1fb_ref,
                f2t_ref, f2b_ref, f2ft_ref, f2fb_ref,
                fow_ref, fob_ref, scal_ref, field_ref):
    def dot(a, b):
        return jax.lax.dot_general(a, b, (((1,), (0,)), ((), ())),
                                   preferred_element_type=jnp.float32)
    mu = muf_ref[0]
    den = denf_ref[0]
    g_geo = _leaky((m_ref[0] - mu) / den * fg_ref[...] + fb_ref[...])
    c = jnp.maximum(dot(cond_ref[0], c1t_ref[...]) + c1b_ref[...], 0.0)
    c = dot(c, c2t_ref[...]) + c2b_ref[...]
    g = jnp.concatenate([g_geo, c], axis=1)
    s = _leaky(dot(g, s1t_ref[...]) + s1b_ref[...])
    scal_ref[0] = dot(s, s2t_ref[...]) + s2b_ref[...]

    q = q_ref[0]
    b1 = dot(g, f1ft_ref[...]) + f1fb_ref[...]
    b2 = dot(g, f2ft_ref[...]) + f2fb_ref[...]
    h1 = jax.lax.dot_general(q, f1t_ref[...], (((0,), (0,)), ((), ())),
                             preferred_element_type=jnp.float32)
    q1 = jnp.sin(SIREN_OMEGA * (h1 + f1b_ref[...]) + b1)
    q2 = jnp.sin(SIREN_OMEGA * (dot(q1, f2t_ref[...]) + f2b_ref[...]) + b2)
    f = jax.lax.dot_general(fow_ref[...], q2, (((1,), (1,)), ((), ())),
                            preferred_element_type=jnp.float32)
    field_ref[0] = f + fob_ref[...].reshape(-1, 1)


def _heads(m, muf, denf, cond, q, p):
    B = m.shape[0]
    M = q.shape[2]
    full = lambda shape: pl.BlockSpec(shape, lambda b: tuple(0 for _ in shape))
    perb = lambda shape: pl.BlockSpec(
        shape, lambda b, _n=len(shape): (b,) + tuple(0 for _ in range(_n - 1)))
    args = [
        (m.reshape(B, 1, 512), perb((1, 1, 512))),
        (muf.reshape(B, 1, 1), perb((1, 1, 1))),
        (denf.reshape(B, 1, 1), perb((1, 1, 1))),
        (p['fuse_g'].reshape(1, 512), full((1, 512))),
        (p['fuse_b'].reshape(1, 512), full((1, 512))),
        (cond.reshape(B, 1, 2), perb((1, 1, 2))),
        (p['c1_W'].T, full((2, 64))),
        (p['c1_b'].reshape(1, 64), full((1, 64))),
        (p['c2_W'].T, full((64, 64))),
        (p['c2_b'].reshape(1, 64), full((1, 64))),
        (p['s1_W'].T, full((576, 256))),
        (p['s1_b'].reshape(1, 256), full((1, 256))),
        (p['s2_W'].T, full((256, 2))),
        (p['s2_b'].reshape(1, 2), full((1, 2))),
        (q, perb((1, 63, M))),
        (p['f1_W'].T, full((63, 256))),
        (p['f1_b'].reshape(1, 256), full((1, 256))),
        (p['f1_fW'].T, full((576, 256))),
        (p['f1_fb'].reshape(1, 256), full((1, 256))),
        (p['f2_W'].T, full((256, 256))),
        (p['f2_b'].reshape(1, 256), full((1, 256))),
        (p['f2_fW'].T, full((576, 256))),
        (p['f2_fb'].reshape(1, 256), full((1, 256))),
        (p['fo_W'], full((4, 256))),
        (p['fo_b'].reshape(1, 4), full((1, 4))),
    ]
    return pl.pallas_call(
        _heads_body,
        grid=(B,),
        in_specs=[a[1] for a in args],
        out_specs=[
            pl.BlockSpec((1, 1, 2), lambda b: (b, 0, 0)),
            pl.BlockSpec((1, 4, M), lambda b: (b, 0, 0)),
        ],
        out_shape=[
            jax.ShapeDtypeStruct((B, 1, 2), jnp.float32),
            jax.ShapeDtypeStruct((B, 4, M), jnp.float32),
        ],
    )(*[a[0] for a in args])


def _pad_cols(a, cp):
    c = a.shape[-1]
    if c == cp:
        return a
    return jnp.pad(a, ((0, 0), (0, cp - c)))


def _edge_layer(xcn, W, gamma, beta):
    B, C, N = xcn.shape
    O = W.shape[0]
    cp = max(16, C)
    scores = _neg_dist(xcn).reshape(B * N, N)
    xt = _pad_cols(jnp.transpose(xcn, (0, 2, 1)).reshape(B * N, C), cp)
    nbr = _sc_topk_gather(scores, xt, n_pts=N, cp=cp)

    wa = jnp.zeros((cp, O), jnp.float32).at[:C].set(W[:, :C].T)
    wb = jnp.zeros((cp, O), jnp.float32).at[:C].set(W[:, C:].T)

    hmax, s1p, s2p = _conv(nbr, wa, wb)
    cnt = O * N * KNN_K
    s1 = s1p.reshape(B, -1).sum(axis=1)
    s2 = s2p.reshape(B, -1).sum(axis=1)
    mean = s1 / cnt
    var = s2 / cnt - mean * mean
    den = jnp.sqrt(var + 1e-5)
    return _norm(hmax, mean, den, gamma, beta, B)


def kernel(x, cond, qxyz, params):
    p = params
    B, _, N = x.shape
    x1 = _edge_layer(x, p['ec1_W'], p['ec1_g'], p['ec1_b'])
    x2 = _edge_layer(jnp.transpose(x1, (0, 2, 1)),
                     p['ec2_W'], p['ec2_g'], p['ec2_b'])
    x3 = _edge_layer(jnp.transpose(x2, (0, 2, 1)),
                     p['ec3_W'], p['ec3_g'], p['ec3_b'])
    cat = jnp.concatenate([x1, x2, x3], axis=2).reshape(B * N, 256)

    mblk, s1p, s2p = _fuse(cat, p['fuse_W'].T)
    GB = mblk.shape[0] // B
    m = jnp.max(mblk.reshape(B, GB, 512), axis=1)
    cntf = 512 * N
    s1 = s1p.reshape(B, GB).sum(axis=1)
    s2 = s2p.reshape(B, GB).sum(axis=1)
    muf = s1 / cntf
    varf = s2 / cntf - muf * muf
    denf = jnp.sqrt(varf + 1e-5)

    Bq = qxyz.shape[0]
    freqs = jnp.pi * (2.0 ** jnp.arange(N_FOURIER, dtype=jnp.float32))
    xs = qxyz[..., None] * freqs.reshape(1, 1, 1, -1)
    sins = jnp.sin(xs).reshape(Bq, 3 * N_FOURIER, -1)
    coss = jnp.cos(xs).reshape(Bq, 3 * N_FOURIER, -1)
    q = jnp.concatenate([qxyz, sins, coss], axis=1)

    scalars, field = _heads(m, muf, denf, cond, q, p)
    return (scalars.reshape(B, 2), field)


# speedup vs baseline: 6.0777x; 1.1289x over previous
import functools
import math

import jax
import jax.numpy as jnp
from jax import lax
from jax.experimental import pallas as pl
from jax.experimental.pallas import tpu as pltpu
from jax.experimental.pallas import tpu_sc as plsc

KNN_K = 20
SIREN_OMEGA = 30.0
N_FOURIER = 10
NEG = float('-inf')

try:
    _SC = plsc.get_sparse_core_info()
    _NC, _NS = _SC.num_cores, _SC.num_subcores
except Exception:
    _NC, _NS = 2, 16
_NW = _NC * _NS


def _leaky(z):
    return jnp.where(z >= 0, z, 0.2 * z)


def _dist_body(x_row_ref, x_all_ref, out_ref):
    xr = x_row_ref[0]
    xa = x_all_ref[0]
    inner = jax.lax.dot_general(xr, xa, (((0,), (0,)), ((), ())),
                                preferred_element_type=jnp.float32)
    xx_r = jnp.sum(xr * xr, axis=0)
    xx_a = jnp.sum(xa * xa, axis=0)
    out_ref[0] = 2.0 * inner - xx_r[:, None] - xx_a[None, :]


def _neg_dist(x):
    B, C, N = x.shape
    RT = 512
    return pl.pallas_call(
        _dist_body,
        grid=(B, N // RT),
        in_specs=[
            pl.BlockSpec((1, C, RT), lambda b, r: (b, 0, r)),
            pl.BlockSpec((1, C, N), lambda b, r: (b, 0, 0)),
        ],
        out_specs=pl.BlockSpec((1, RT, N), lambda b, r: (b, r, 0)),
        out_shape=jax.ShapeDtypeStruct((B, N, N), jnp.float32),
    )(x, x)


def _lane_splat_f32(vec, lane, iota):
    m = jnp.where(iota == lane, vec, jnp.full((16,), NEG, jnp.float32))
    return jnp.full((16,), jnp.max(m))


NBR = 24


@functools.partial(jax.jit, static_argnames=('n_pts', 'cp'))
def _sc_topk_gather(scores, xt, n_pts, cp):
    BN, N = scores.shape
    K = KNN_K
    rows_per_w = BN // _NW
    mesh = plsc.VectorSubcoreMesh(core_axis_name="c", subcore_axis_name="s")

    @functools.partial(
        pl.kernel,
        out_type=jax.ShapeDtypeStruct((BN, NBR, cp), jnp.float32),
        mesh=mesh,
        compiler_params=pltpu.CompilerParams(needs_layout_passes=False,
                                             use_tc_tiling_on_sc=False),
        scratch_types=[
            pltpu.VMEM((2, 1, N), jnp.float32),
            pltpu.VMEM((N,), jnp.int32),
            pltpu.VMEM((NBR,), jnp.int32),
            pltpu.VMEM((NBR, cp), jnp.float32),
            pltpu.SemaphoreType.DMA,
            pltpu.SemaphoreType.DMA,
        ],
    )
    def body(scores_hbm, xt_hbm, nbr_hbm, row_v, cand_v, sel_v, rows_v,
             rsem, gsem):
        wid = lax.axis_index("s") * _NC + lax.axis_index("c")
        base = wid * rows_per_w
        bN = (base // n_pts) * n_pts
        iota = lax.iota(jnp.int32, 16)
        nvecs = N // 16
        neg_v = jnp.full((16,), NEG, jnp.float32)

        pltpu.async_copy(scores_hbm.at[pl.ds(base, 1)], row_v.at[0], rsem)

        def do_row(r, _carry):
            i = base + r
            buf = lax.rem(r, 2)
            pltpu.make_async_copy(
                scores_hbm.at[pl.ds(i, 1)], row_v.at[buf], rsem).wait()

            @pl.when(r + 1 < rows_per_w)
            def _prefetch():
                pltpu.async_copy(scores_hbm.at[pl.ds(i + 1, 1)],
                                 row_v.at[1 - buf], rsem)

            def amax(h):
                def st(j, acc):
                    return jnp.maximum(
                        acc, row_v[buf, 0, pl.ds(h * (N // 2) + j * 16, 16)])
                return lax.fori_loop(0, nvecs // 2, st, neg_v)
            acc0 = amax(0)
            acc1 = amax(1)
            s0, _ = plsc.sort_key_val(acc0, iota, descending=True)
            s1, _ = plsc.sort_key_val(acc1, iota, descending=True)
            hmin = jnp.minimum(s0, jnp.flip(s1, 0))
            sh, _ = plsc.sort_key_val(hmin, iota, descending=True)
            t_lb = _lane_splat_f32(sh, 3, iota)

            def bstep(j, off):
                v = row_v[buf, 0, pl.ds(j * 16, 16)]
                m = v >= t_lb
                csum = plsc.cumsum(m.astype(jnp.int32))
                pos = off + csum - 1
                plsc.store_scatter(cand_v, [pos], j * 16 + iota, mask=m)
                return off + plsc.all_reduce_population_count(m)
            offv = lax.fori_loop(0, nvecs, bstep,
                                 jnp.zeros((16,), jnp.int32))
            ncand = jnp.max(offv)
            nvec_c = (ncand + 15) // 16
            ncand_v = jnp.full((16,), ncand)
            zero16 = jnp.zeros((16,), jnp.int32)

            def load_cand(v):
                idxs = cand_v[pl.ds(v * 16, 16)]
                ok = (v * 16 + iota) < ncand_v
                idxs = jnp.where(ok, idxs, zero16)
                vals = plsc.load_gather(
                    row_v, [jnp.full((16,), buf), zero16, idxs])
                return idxs, jnp.where(ok, vals, neg_v)

            def cstep(v, carry):
                T0, T1 = carry
                _, vals = load_cand(v)
                s, _ = plsc.sort_key_val(vals, iota, descending=True)
                u0 = jnp.maximum(T1, jnp.flip(s, 0))
                su, _ = plsc.sort_key_val(u0, iota, descending=True)
                ru = jnp.flip(su, 0)
                n0, _ = plsc.sort_key_val(jnp.maximum(T0, ru), iota,
                                          descending=True)
                n1, _ = plsc.sort_key_val(jnp.minimum(T0, ru), iota,
                                          descending=True)
                return n0, n1
            _, T1 = lax.fori_loop(0, nvec_c, cstep, (neg_v, neg_v))
            t20 = _lane_splat_f32(T1, 3, iota)

            self_v = jnp.full((16,), i)
            sel_v[pl.ds(0, 16)] = self_v
            sel_v[pl.ds(NBR - 16, 16)] = self_v
            cap = jnp.full((16,), NBR)

            def d1(v, cnt):
                idxs, vals = load_cand(v)
                m = vals > t20
                pos = cnt + plsc.cumsum(m.astype(jnp.int32)) - 1
                keep = m & (pos < cap)
                plsc.store_scatter(sel_v, [pos], bN + idxs, mask=keep)
                return cnt + plsc.all_reduce_population_count(keep)
            ngt = lax.fori_loop(0, nvec_c, d1, jnp.zeros((16,), jnp.int32))

            def d2(v, cnt):
                idxs, vals = load_cand(v)
                m = vals == t20
                c = cnt + plsc.cumsum(m.astype(jnp.int32))
                keep = m & (c <= K)
                plsc.store_scatter(sel_v, [c - 1], bN + idxs, mask=keep)
                return cnt + plsc.all_reduce_population_count(keep)
            lax.fori_loop(0, nvec_c, d2, ngt)

            pltpu.async_copy(xt_hbm.at[sel_v], rows_v, gsem).wait()
            pltpu.sync_copy(rows_v, nbr_hbm.at[i])
            return _carry

        lax.fori_loop(0, rows_per_w, do_row, jnp.int32(0))

    return body(scores, xt)


def _conv_body(nbr_ref, wa_ref, wb_ref, hmax_ref, s1_ref, s2_ref):
    xi = nbr_ref[:, KNN_K, :]
    a = jax.lax.dot_general(xi, wa_ref[...], (((1,), (0,)), ((), ())),
                            preferred_element_type=jnp.float32)
    wb = wb_ref[...]
    acc = None
    s1 = jnp.float32(0.0)
    s2 = jnp.float32(0.0)
    for kk in range(KNN_K):
        h = a + jax.lax.dot_general(
            nbr_ref[:, kk, :] - xi, wb, (((1,), (0,)), ((), ())),
            preferred_element_type=jnp.float32)
        s1 = s1 + jnp.sum(h)
        s2 = s2 + jnp.sum(h * h)
        acc = h if acc is None else jnp.maximum(acc, h)
    hmax_ref[...] = acc
    s1_ref[...] = s1.reshape(1, 1, 1)
    s2_ref[...] = s2.reshape(1, 1, 1)


def _conv(nbr, wa, wb):
    BN, NB, Cp = nbr.shape
    O = wa.shape[1]
    P = 512
    G = BN // P
    return pl.pallas_call(
        _conv_body,
        grid=(G,),
        in_specs=[
            pl.BlockSpec((P, NB, Cp), lambda g: (g, 0, 0)),
            pl.BlockSpec((Cp, O), lambda g: (0, 0)),
            pl.BlockSpec((Cp, O), lambda g: (0, 0)),
        ],
        out_specs=[
            pl.BlockSpec((P, O), lambda g: (g, 0)),
            pl.BlockSpec((1, 1, 1), lambda g: (g, 0, 0)),
            pl.BlockSpec((1, 1, 1), lambda g: (g, 0, 0)),
        ],
        out_shape=[
            jax.ShapeDtypeStruct((BN, O), jnp.float32),
            jax.ShapeDtypeStruct((G, 1, 1), jnp.float32),
            jax.ShapeDtypeStruct((G, 1, 1), jnp.float32),
        ],
    )(nbr, wa, wb)


def _norm_body(h_ref, mu_ref, den_ref, g_ref, b_ref, o_ref):
    mu = mu_ref[0]
    den = den_ref[0]
    xn = (h_ref[0] - mu) / den
    o_ref[0] = _leaky(xn * g_ref[...] + b_ref[...])


def _norm(h, mu, den, gamma, beta, B):
    BN, O = h.shape
    N = BN // B
    P = 1024
    h3 = h.reshape(B, N, O)
    return pl.pallas_call(
        _norm_body,
        grid=(B, N // P),
        in_specs=[
            pl.BlockSpec((1, P, O), lambda b, g: (b, g, 0)),
            pl.BlockSpec((1, 1, 1), lambda b, g: (b, 0, 0)),
            pl.BlockSpec((1, 1, 1), lambda b, g: (b, 0, 0)),
            pl.BlockSpec((1, O), lambda b, g: (0, 0)),
            pl.BlockSpec((1, O), lambda b, g: (0, 0)),
        ],
        out_specs=pl.BlockSpec((1, P, O), lambda b, g: (b, g, 0)),
        out_shape=jax.ShapeDtypeStruct((B, N, O), jnp.float32),
    )(h3, mu.reshape(B, 1, 1), den.reshape(B, 1, 1),
      gamma.reshape(1, O), beta.reshape(1, O))


def _fuse_body(c_ref, w_ref, m_ref, s1_ref, s2_ref):
    h = jax.lax.dot_general(c_ref[...], w_ref[...], (((1,), (0,)), ((), ())),
                            preferred_element_type=jnp.float32)
    m_ref[...] = jnp.max(h, axis=0).reshape(1, 1, -1)
    s1_ref[...] = jnp.sum(h).reshape(1, 1, 1)
    s2_ref[...] = jnp.sum(h * h).reshape(1, 1, 1)


def _fuse(cat, wt):
    BN, C = cat.shape
    O = wt.shape[1]
    P = 512
    G = BN // P
    return pl.pallas_call(
        _fuse_body,
        grid=(G,),
        in_specs=[
            pl.BlockSpec((P, C), lambda g: (g, 0)),
            pl.BlockSpec((C, O), lambda g: (0, 0)),
        ],
        out_specs=[
            pl.BlockSpec((1, 1, O), lambda g: (g, 0, 0)),
            pl.BlockSpec((1, 1, 1), lambda g: (g, 0, 0)),
            pl.BlockSpec((1, 1, 1), lambda g: (g, 0, 0)),
        ],
        out_shape=[
            jax.ShapeDtypeStruct((G, 1, O), jnp.float32),
            jax.ShapeDtypeStruct((G, 1, 1), jnp.float32),
            jax.ShapeDtypeStruct((G, 1, 1), jnp.float32),
        ],
    )(cat, wt)


def _heads_body(m_ref, muf_ref, denf_ref, fg_ref, fb_ref, cond_ref,
                c1t_ref, c1b_ref, c2t_ref, c2b_ref,
                s1t_ref, s1b_ref, s2t_ref, s2b_ref,
                q_ref, f1t_ref, f1b_ref, f1ft_ref, f1fb_ref,
                f2t_ref, f2b_ref, f2ft_ref, f2fb_ref,
                fow_ref, fob_ref, scal_ref, field_ref):
    def dot(a, b):
        return jax.lax.dot_general(a, b, (((1,), (0,)), ((), ())),
                                   preferred_element_type=jnp.float32)
    mu = muf_ref[0]
    den = denf_ref[0]
    g_geo = _leaky((m_ref[0] - mu) / den * fg_ref[...] + fb_ref[...])
    c = jnp.maximum(dot(cond_ref[0], c1t_ref[...]) + c1b_ref[...], 0.0)
    c = dot(c, c2t_ref[...]) + c2b_ref[...]
    g = jnp.concatenate([g_geo, c], axis=1)
    s = _leaky(dot(g, s1t_ref[...]) + s1b_ref[...])
    scal_ref[0] = dot(s, s2t_ref[...]) + s2b_ref[...]

    q = q_ref[0]
    b1 = dot(g, f1ft_ref[...]) + f1fb_ref[...]
    b2 = dot(g, f2ft_ref[...]) + f2fb_ref[...]
    h1 = jax.lax.dot_general(q, f1t_ref[...], (((0,), (0,)), ((), ())),
                             preferred_element_type=jnp.float32)
    q1 = jnp.sin(SIREN_OMEGA * (h1 + f1b_ref[...]) + b1)
    q2 = jnp.sin(SIREN_OMEGA * (dot(q1, f2t_ref[...]) + f2b_ref[...]) + b2)
    f = jax.lax.dot_general(fow_ref[...], q2, (((1,), (1,)), ((), ())),
                            preferred_element_type=jnp.float32)
    field_ref[0] = f + fob_ref[...].reshape(-1, 1)


def _heads(m, muf, denf, cond, q, p):
    B = m.shape[0]
    M = q.shape[2]
    full = lambda shape: pl.BlockSpec(shape, lambda b: tuple(0 for _ in shape))
    perb = lambda shape: pl.BlockSpec(
        shape, lambda b, _n=len(shape): (b,) + tuple(0 for _ in range(_n - 1)))
    args = [
        (m.reshape(B, 1, 512), perb((1, 1, 512))),
        (muf.reshape(B, 1, 1), perb((1, 1, 1))),
        (denf.reshape(B, 1, 1), perb((1, 1, 1))),
        (p['fuse_g'].reshape(1, 512), full((1, 512))),
        (p['fuse_b'].reshape(1, 512), full((1, 512))),
        (cond.reshape(B, 1, 2), perb((1, 1, 2))),
        (p['c1_W'].T, full((2, 64))),
        (p['c1_b'].reshape(1, 64), full((1, 64))),
        (p['c2_W'].T, full((64, 64))),
        (p['c2_b'].reshape(1, 64), full((1, 64))),
        (p['s1_W'].T, full((576, 256))),
        (p['s1_b'].reshape(1, 256), full((1, 256))),
        (p['s2_W'].T, full((256, 2))),
        (p['s2_b'].reshape(1, 2), full((1, 2))),
        (q, perb((1, 63, M))),
        (p['f1_W'].T, full((63, 256))),
        (p['f1_b'].reshape(1, 256), full((1, 256))),
        (p['f1_fW'].T, full((576, 256))),
        (p['f1_fb'].reshape(1, 256), full((1, 256))),
        (p['f2_W'].T, full((256, 256))),
        (p['f2_b'].reshape(1, 256), full((1, 256))),
        (p['f2_fW'].T, full((576, 256))),
        (p['f2_fb'].reshape(1, 256), full((1, 256))),
        (p['fo_W'], full((4, 256))),
        (p['fo_b'].reshape(1, 4), full((1, 4))),
    ]
    return pl.pallas_call(
        _heads_body,
        grid=(B,),
        in_specs=[a[1] for a in args],
        out_specs=[
            pl.BlockSpec((1, 1, 2), lambda b: (b, 0, 0)),
            pl.BlockSpec((1, 4, M), lambda b: (b, 0, 0)),
        ],
        out_shape=[
            jax.ShapeDtypeStruct((B, 1, 2), jnp.float32),
            jax.ShapeDtypeStruct((B, 4, M), jnp.float32),
        ],
    )(*[a[0] for a in args])


def _pad_cols(a, cp):
    c = a.shape[-1]
    if c == cp:
        return a
    return jnp.pad(a, ((0, 0), (0, cp - c)))


def _edge_layer(xcn, W, gamma, beta):
    B, C, N = xcn.shape
    O = W.shape[0]
    cp = max(16, C)
    scores = _neg_dist(xcn).reshape(B * N, N)
    xt = _pad_cols(jnp.transpose(xcn, (0, 2, 1)).reshape(B * N, C), cp)
    nbr = _sc_topk_gather(scores, xt, n_pts=N, cp=cp)

    wa = jnp.zeros((cp, O), jnp.float32).at[:C].set(W[:, :C].T)
    wb = jnp.zeros((cp, O), jnp.float32).at[:C].set(W[:, C:].T)

    hmax, s1p, s2p = _conv(nbr, wa, wb)
    cnt = O * N * KNN_K
    s1 = s1p.reshape(B, -1).sum(axis=1)
    s2 = s2p.reshape(B, -1).sum(axis=1)
    mean = s1 / cnt
    var = s2 / cnt - mean * mean
    den = jnp.sqrt(var + 1e-5)
    return _norm(hmax, mean, den, gamma, beta, B)


def kernel(x, cond, qxyz, params):
    p = params
    B, _, N = x.shape
    x1 = _edge_layer(x, p['ec1_W'], p['ec1_g'], p['ec1_b'])
    x2 = _edge_layer(jnp.transpose(x1, (0, 2, 1)),
                     p['ec2_W'], p['ec2_g'], p['ec2_b'])
    x3 = _edge_layer(jnp.transpose(x2, (0, 2, 1)),
                     p['ec3_W'], p['ec3_g'], p['ec3_b'])
    cat = jnp.concatenate([x1, x2, x3], axis=2).reshape(B * N, 256)

    mblk, s1p, s2p = _fuse(cat, p['fuse_W'].T)
    GB = mblk.shape[0] // B
    m = jnp.max(mblk.reshape(B, GB, 512), axis=1)
    cntf = 512 * N
    s1 = s1p.reshape(B, GB).sum(axis=1)
    s2 = s2p.reshape(B, GB).sum(axis=1)
    muf = s1 / cntf
    varf = s2 / cntf - muf * muf
    denf = jnp.sqrt(varf + 1e-5)

    Bq = qxyz.shape[0]
    freqs = jnp.pi * (2.0 ** jnp.arange(N_FOURIER, dtype=jnp.float32))
    xs = qxyz[..., None] * freqs.reshape(1, 1, 1, -1)
    sins = jnp.sin(xs).reshape(Bq, 3 * N_FOURIER, -1)
    coss = jnp.cos(xs).reshape(Bq, 3 * N_FOURIER, -1)
    q = jnp.concatenate([qxyz, sins, coss], axis=1)

    scalars, field = _heads(m, muf, denf, cond, q, p)
    return (scalars.reshape(B, 2), field)
